# speedup vs baseline: 7.3137x; 7.3137x over previous
import functools

import jax
import jax.numpy as jnp
from jax import lax
from jax.experimental import pallas as pl
from jax.experimental.pallas import tpu as pltpu
from jax.experimental.pallas import tpu_sc as plsc

N_NODES = 10000
D_FEAT = 128
N_EDGES = 320000

NCORES = 2
NSUB = 16
NTILES = NCORES * NSUB
LANES = 16

CH = 128
NPAD = 10240
EPAD = 327680
E_TILE = EPAD // NTILES
NCHUNK = E_TILE // CH
ROWS_TILE = NPAD // NSUB
RB = 10
TCB = NPAD // RB

_MESH = plsc.VectorSubcoreMesh(core_axis_name="c", subcore_axis_name="s")


def _sc_deg(dst_pad):
    @functools.partial(
        pl.kernel,
        out_type=jax.ShapeDtypeStruct((NCORES, NPAD, LANES), jnp.float32),
        mesh=_MESH,
        scratch_types=[
            pltpu.VMEM((CH,), jnp.int32),
            pltpu.VMEM((CH, LANES), jnp.float32),
            pltpu.VMEM_SHARED((NPAD, LANES), jnp.float32),
        ],
    )
    def deg_kernel(dst_hbm, out_hbm, idx_v, ones_v, acc_sh):
        c = lax.axis_index("c")
        s = lax.axis_index("s")
        wid = c * NSUB + s

        @pl.loop(0, CH)
        def _(i):
            ones_v[i, :] = jnp.zeros((LANES,), jnp.float32)

        for r in range(ROWS_TILE // CH):
            pltpu.sync_copy(ones_v, acc_sh.at[pl.ds(s * ROWS_TILE + r * CH, CH)])

        @pl.loop(0, CH)
        def _(i):
            ones_v[i, :] = jnp.ones((LANES,), jnp.float32)

        plsc.subcore_barrier()

        base = wid * E_TILE

        @pl.loop(0, NCHUNK)
        def _(k):
            pltpu.sync_copy(dst_hbm.at[pl.ds(base + k * CH, CH)], idx_v)
            pltpu.sync_copy(ones_v, acc_sh.at[idx_v], add=True)

        plsc.subcore_barrier()
        pltpu.sync_copy(
            acc_sh.at[pl.ds(s * ROWS_TILE, ROWS_TILE)],
            out_hbm.at[c].at[pl.ds(s * ROWS_TILE, ROWS_TILE)],
        )

    return deg_kernel(dst_pad)


def _sc_agg(g, src_pad, dst_pad):
    @functools.partial(
        pl.kernel,
        out_type=jax.ShapeDtypeStruct((NCORES, NPAD, D_FEAT), jnp.float32),
        mesh=_MESH,
        scratch_types=[
            pltpu.VMEM((CH,), jnp.int32),
            pltpu.VMEM((CH,), jnp.int32),
            pltpu.VMEM((CH, D_FEAT), jnp.float32),
            pltpu.VMEM_SHARED((NPAD, D_FEAT), jnp.float32),
            pltpu.SemaphoreType.DMA,
        ],
    )
    def agg_kernel(g_hbm, src_hbm, dst_hbm, out_hbm, sidx, didx, rows, acc_sh, sem):
        c = lax.axis_index("c")
        s = lax.axis_index("s")
        wid = c * NSUB + s

        @pl.loop(0, CH)
        def _(i):
            for j in range(D_FEAT // LANES):
                rows[i, pl.ds(j * LANES, LANES)] = jnp.zeros((LANES,), jnp.float32)

        for r in range(ROWS_TILE // CH):
            pltpu.sync_copy(rows, acc_sh.at[pl.ds(s * ROWS_TILE + r * CH, CH)])

        plsc.subcore_barrier()

        base = wid * E_TILE

        @pl.loop(0, NCHUNK)
        def _(k):
            pltpu.sync_copy(src_hbm.at[pl.ds(base + k * CH, CH)], sidx)
            pltpu.sync_copy(dst_hbm.at[pl.ds(base + k * CH, CH)], didx)
            pltpu.async_copy(g_hbm.at[sidx], rows, sem).wait()
            pltpu.sync_copy(rows, acc_sh.at[didx], add=True)

        plsc.subcore_barrier()
        pltpu.sync_copy(
            acc_sh.at[pl.ds(s * ROWS_TILE, ROWS_TILE)],
            out_hbm.at[c].at[pl.ds(s * ROWS_TILE, ROWS_TILE)],
        )

    return agg_kernel(g, src_pad, dst_pad)


def _dinv_block(d0_ref, d1_ref):
    deg = d0_ref[0][:, 0:1] + d1_ref[0][:, 0:1] + 1.0
    return lax.rsqrt(deg)


def _tc1(x_pad, W1, degp):
    def body(x_ref, w_ref, d0_ref, d1_ref, g_ref):
        dinv = _dinv_block(d0_ref, d1_ref)
        h = jnp.dot(x_ref[...], w_ref[...],
                    preferred_element_type=jnp.float32,
                    precision=lax.Precision.HIGHEST)
        g_ref[...] = h * dinv

    return pl.pallas_call(
        body,
        grid=(RB,),
        in_specs=[
            pl.BlockSpec((TCB, D_FEAT), lambda i: (i, 0)),
            pl.BlockSpec((D_FEAT, D_FEAT), lambda i: (0, 0)),
            pl.BlockSpec((1, TCB, LANES), lambda i: (0, i, 0)),
            pl.BlockSpec((1, TCB, LANES), lambda i: (1, i, 0)),
        ],
        out_specs=pl.BlockSpec((TCB, D_FEAT), lambda i: (i, 0)),
        out_shape=jax.ShapeDtypeStruct((NPAD, D_FEAT), jnp.float32),
    )(x_pad, W1, degp, degp)


def _tc2(acc1, g1, degp, b1r, W2):
    def body(a0_ref, a1_ref, g_ref, d0_ref, d1_ref, b_ref, w_ref, o_ref):
        dinv = _dinv_block(d0_ref, d1_ref)
        ssum = a0_ref[0] + a1_ref[0] + g_ref[...]
        z = jnp.maximum(ssum * dinv + b_ref[...], 0.0)
        h2 = jnp.dot(z, w_ref[...],
                     preferred_element_type=jnp.float32,
                     precision=lax.Precision.HIGHEST)
        o_ref[...] = h2 * dinv

    return pl.pallas_call(
        body,
        grid=(RB,),
        in_specs=[
            pl.BlockSpec((1, TCB, D_FEAT), lambda i: (0, i, 0)),
            pl.BlockSpec((1, TCB, D_FEAT), lambda i: (1, i, 0)),
            pl.BlockSpec((TCB, D_FEAT), lambda i: (i, 0)),
            pl.BlockSpec((1, TCB, LANES), lambda i: (0, i, 0)),
            pl.BlockSpec((1, TCB, LANES), lambda i: (1, i, 0)),
            pl.BlockSpec((1, D_FEAT), lambda i: (0, 0)),
            pl.BlockSpec((D_FEAT, D_FEAT), lambda i: (0, 0)),
        ],
        out_specs=pl.BlockSpec((TCB, D_FEAT), lambda i: (i, 0)),
        out_shape=jax.ShapeDtypeStruct((NPAD, D_FEAT), jnp.float32),
    )(acc1, acc1, g1, degp, degp, b1r, W2)


def _tc3(acc2, g2, degp, b2r):
    def body(a0_ref, a1_ref, g_ref, d0_ref, d1_ref, b_ref, o_ref):
        dinv = _dinv_block(d0_ref, d1_ref)
        ssum = a0_ref[0] + a1_ref[0] + g_ref[...]
        o_ref[...] = ssum * dinv + b_ref[...]

    return pl.pallas_call(
        body,
        grid=(RB,),
        in_specs=[
            pl.BlockSpec((1, TCB, D_FEAT), lambda i: (0, i, 0)),
            pl.BlockSpec((1, TCB, D_FEAT), lambda i: (1, i, 0)),
            pl.BlockSpec((TCB, D_FEAT), lambda i: (i, 0)),
            pl.BlockSpec((1, TCB, LANES), lambda i: (0, i, 0)),
            pl.BlockSpec((1, TCB, LANES), lambda i: (1, i, 0)),
            pl.BlockSpec((1, D_FEAT), lambda i: (0, 0)),
        ],
        out_specs=pl.BlockSpec((TCB, D_FEAT), lambda i: (i, 0)),
        out_shape=jax.ShapeDtypeStruct((NPAD, D_FEAT), jnp.float32),
    )(acc2, acc2, g2, degp, degp, b2r)


def kernel(x, edge_index, W1, b1, W2, b2):
    ei = edge_index.astype(jnp.int32)
    padn = EPAD - N_EDGES
    src_pad = jnp.concatenate([ei[0], jnp.zeros((padn,), jnp.int32)])
    dst_pad = jnp.concatenate([ei[1], jnp.full((padn,), N_NODES, jnp.int32)])
    x_pad = jnp.concatenate(
        [x, jnp.zeros((NPAD - N_NODES, D_FEAT), jnp.float32)])
    b1r = b1.reshape(1, D_FEAT)
    b2r = b2.reshape(1, D_FEAT)

    degp = _sc_deg(dst_pad)
    g1 = _tc1(x_pad, W1, degp)
    acc1 = _sc_agg(g1, src_pad, dst_pad)
    g2 = _tc2(acc1, g1, degp, b1r, W2)
    acc2 = _sc_agg(g2, src_pad, dst_pad)
    out = _tc3(acc2, g2, degp, b2r)
    return out[:N_NODES]
